# baseline (speedup 1.0000x reference)
import functools

import jax
import jax.numpy as jnp
from jax import lax
from jax.experimental import pallas as pl
from jax.experimental.pallas import tpu as pltpu
from jax.experimental.pallas import tpu_sc as plsc

VOCAB = 1_000_000
D = 64
L = 20
NC = 2
NS = 16
NW = NC * NS
C = 128


def _cbow_body(b_per_w, n_chunks,
               ctxT_hbm, cen_hbm, ctab_hbm, otab_hbm, out_hbm,
               idxT, cidx, buf, acc, cbuf, r0v, score_v,
               sem_a, sem_b, sem_c):
    cid = lax.axis_index("c")
    sid = lax.axis_index("s")
    wid = sid * NC + cid
    base = wid * b_per_w

    pltpu.sync_copy(ctab_hbm.at[0], r0v)

    for k in range(n_chunks):
        cb = base + k * C

        pltpu.sync_copy(ctxT_hbm.at[:, pl.ds(cb, C)], idxT)
        pltpu.sync_copy(cen_hbm.at[pl.ds(cb, C)], cidx)
        ce_cp = pltpu.async_copy(otab_hbm.at[cidx], cbuf, sem_c)

        def zero_body(r, _):
            for c4 in range(D // 16):
                acc[r, pl.ds(c4 * 16, 16)] = jnp.zeros((16,), jnp.float32)
            return 0
        lax.fori_loop(0, C, zero_body, 0)

        cps = [None, None]
        cps[0] = pltpu.async_copy(ctab_hbm.at[idxT.at[0]], buf.at[0], sem_a)
        for j in range(L):
            p = j % 2
            if j + 1 < L:
                pn = (j + 1) % 2
                cps[pn] = pltpu.async_copy(
                    ctab_hbm.at[idxT.at[j + 1]], buf.at[pn],
                    sem_b if pn else sem_a)
            cps[p].wait()

            def acc_body(r, _, p=p):
                for c4 in range(D // 16):
                    sl = pl.ds(c4 * 16, 16)
                    plsc.addupdate(acc.at[r, sl], buf[p, r, sl])
                return 0
            lax.fori_loop(0, C, acc_body, 0)

        ce_cp.wait()

        def grp_body(g, _):
            rows = g * 16 + lax.iota(jnp.int32, 16)

            nz = jnp.zeros((16,), jnp.float32)
            for j in range(L):
                colj = idxT[j, pl.ds(g * 16, 16)]
                nz = nz + jnp.where(colj == 0, 1.0, 0.0).astype(jnp.float32)

            def d_body(d, carry):
                A, Bv = carry
                dsp = jnp.full((16,), d, jnp.int32)
                colA = plsc.load_gather(acc, [rows, dsp])
                colC = plsc.load_gather(cbuf, [rows, dsp])
                r0d = plsc.load_gather(r0v, [dsp])
                return A + colA * colC, Bv + r0d * colC

            zero = jnp.zeros((16,), jnp.float32)
            A, Bv = lax.fori_loop(0, D, d_body, (zero, zero))
            cnt = jnp.float32(L) - nz
            sc = (A - nz * Bv) / (cnt + 1e-8)
            sc = jnp.where(nz >= jnp.float32(L), 0.0, sc)
            score_v[pl.ds(g * 16, 16)] = sc
            return 0
        lax.fori_loop(0, C // 16, grp_body, 0)

        pltpu.sync_copy(score_v, out_hbm.at[pl.ds(cb, C)])


@jax.jit
def _cbow_sc(ctxT, center, context_table, output_table):
    B = ctxT.shape[1]
    b_per_w = B // NW
    n_chunks = b_per_w // C
    mesh = plsc.VectorSubcoreMesh(core_axis_name="c", subcore_axis_name="s")

    kern = pl.kernel(
        functools.partial(_cbow_body, b_per_w, n_chunks),
        out_type=jax.ShapeDtypeStruct((B,), jnp.float32),
        mesh=mesh,
        compiler_params=pltpu.CompilerParams(
            needs_layout_passes=False, use_tc_tiling_on_sc=False),
        scratch_types=[
            pltpu.VMEM((L, C), jnp.int32),
            pltpu.VMEM((C,), jnp.int32),
            pltpu.VMEM((2, C, D), jnp.float32),
            pltpu.VMEM((C, D), jnp.float32),
            pltpu.VMEM((C, D), jnp.float32),
            pltpu.VMEM((D,), jnp.float32),
            pltpu.VMEM((C,), jnp.float32),
            pltpu.SemaphoreType.DMA,
            pltpu.SemaphoreType.DMA,
            pltpu.SemaphoreType.DMA,
        ],
    )
    return kern(ctxT, center, context_table, output_table)


def kernel(context_words, center, context_table, output_table):
    ctxT = context_words.astype(jnp.int32).T
    return _cbow_sc(ctxT, center.astype(jnp.int32),
                    context_table, output_table)
